# baseline (speedup 1.0000x reference)
import functools

import jax
import jax.numpy as jnp
from jax import lax
from jax.experimental import pallas as pl
from jax.experimental.pallas import tpu as pltpu
from jax.experimental.pallas import tpu_sc as plsc

B = 16384
K = 128
NW = 32
PW = B // NW
NCHUNK = 4
C = PW // NCHUNK
G = C // 16


def _mf_body(u_hbm, i_hbm, uf_hbm, if_hbm, ub_hbm, ib_hbm, out_hbm,
             uidx0, uidx1, uidx2, uidx3, iidx0, iidx1, iidx2, iidx3,
             ubuf0, ubuf1, ibuf0, ibuf1, buv, biv, outv,
             sem_idx, sem_g0, sem_g1, sem_b):
    wid = lax.axis_index("s") * 2 + lax.axis_index("c")
    base = wid * PW
    uidx = [uidx0, uidx1, uidx2, uidx3]
    iidx = [iidx0, iidx1, iidx2, iidx3]
    ubuf = [ubuf0, ubuf1]
    ibuf = [ibuf0, ibuf1]

    idx_cp = []
    for t in range(NCHUNK):
        idx_cp.append(pltpu.async_copy(u_hbm.at[wid, t], uidx[t], sem_idx))
        idx_cp.append(pltpu.async_copy(i_hbm.at[wid, t], iidx[t], sem_idx))
    for cp in idx_cp:
        cp.wait()

    b_cp = []
    for t in range(NCHUNK):
        b_cp.append(pltpu.async_copy(ub_hbm.at[uidx[t]],
                                     buv.at[pl.ds(t * C, C)], sem_b))
        b_cp.append(pltpu.async_copy(ib_hbm.at[iidx[t]],
                                     biv.at[pl.ds(t * C, C)], sem_b))

    def fire(t):
        sem = sem_g0 if t % 2 == 0 else sem_g1
        return [pltpu.async_copy(uf_hbm.at[uidx[t]], ubuf[t % 2], sem),
                pltpu.async_copy(if_hbm.at[iidx[t]], ibuf[t % 2], sem)]

    def compute(t):
        ub_, ib_ = ubuf[t % 2], ibuf[t % 2]

        def group(g, carry):
            for l in range(16):
                p = g * 16 + l
                acc = ub_[p, pl.ds(0, 16)] * ib_[p, pl.ds(0, 16)]
                for j in range(1, 8):
                    acc = acc + (ub_[p, pl.ds(j * 16, 16)]
                                 * ib_[p, pl.ds(j * 16, 16)])
                pos = jnp.full((16,), t * C + g * 16 + l, jnp.int32)
                plsc.addupdate_scatter(outv, [pos], acc)
            return carry

        lax.fori_loop(0, G, group, 0)

    inflight = fire(0)

    for cp in b_cp:
        cp.wait()

    def bias_group(g, carry):
        s = pl.ds(g * 16, 16)
        outv[s] = buv[s] + biv[s]
        return carry

    lax.fori_loop(0, PW // 16, bias_group, 0)

    for t in range(NCHUNK):
        nxt = fire(t + 1) if t + 1 < NCHUNK else []
        for cp in inflight:
            cp.wait()
        compute(t)
        inflight = nxt

    pltpu.sync_copy(outv, out_hbm.at[pl.ds(base, PW)])


@jax.jit
def kernel(u, i, user_f, item_f, user_b, item_b, global_b):
    u2 = u.astype(jnp.int32).reshape(NW, NCHUNK, C)
    i2 = i.astype(jnp.int32).reshape(NW, NCHUNK, C)
    ub = user_b.reshape(-1)
    ib = item_b.reshape(-1)
    mesh = plsc.VectorSubcoreMesh(core_axis_name="c", subcore_axis_name="s")
    fn = pl.kernel(
        _mf_body,
        out_type=jax.ShapeDtypeStruct((B,), jnp.float32),
        mesh=mesh,
        scratch_types=[pltpu.VMEM((C,), jnp.int32)] * 8
        + [pltpu.VMEM((C, K), jnp.float32)] * 4
        + [pltpu.VMEM((PW,), jnp.float32)] * 3
        + [pltpu.SemaphoreType.DMA] * 4,
        compiler_params=pltpu.CompilerParams(needs_layout_passes=False),
    )
    out = fn(u2, i2, user_f, item_f, ub, ib)
    return out + global_b


# speedup vs baseline: 1.0436x; 1.0436x over previous
import functools

import jax
import jax.numpy as jnp
from jax import lax
from jax.experimental import pallas as pl
from jax.experimental.pallas import tpu as pltpu
from jax.experimental.pallas import tpu_sc as plsc

B = 16384
K = 128
NW = 32
PW = B // NW
NCHUNK = 4
C = PW // NCHUNK
G = C // 16


def _mf_body(u_hbm, i_hbm, uf_hbm, if_hbm, ub_hbm, ib_hbm, out_hbm,
             uidx0, uidx1, uidx2, uidx3, iidx0, iidx1, iidx2, iidx3,
             ubuf0, ubuf1, ibuf0, ibuf1, buv, biv, outv,
             sem_idx, sem_g0, sem_g1, sem_b):
    wid = lax.axis_index("s") * 2 + lax.axis_index("c")
    base = wid * PW
    uidx = [uidx0, uidx1, uidx2, uidx3]
    iidx = [iidx0, iidx1, iidx2, iidx3]
    ubuf = [ubuf0, ubuf1]
    ibuf = [ibuf0, ibuf1]

    idx_cp = []
    for t in range(NCHUNK):
        idx_cp.append(pltpu.async_copy(u_hbm.at[wid, t], uidx[t], sem_idx))
        idx_cp.append(pltpu.async_copy(i_hbm.at[wid, t], iidx[t], sem_idx))
    for cp in idx_cp:
        cp.wait()

    b_cp = []
    for t in range(NCHUNK):
        b_cp.append(pltpu.async_copy(ub_hbm.at[uidx[t]],
                                     buv.at[pl.ds(t * C, C)], sem_b))
        b_cp.append(pltpu.async_copy(ib_hbm.at[iidx[t]],
                                     biv.at[pl.ds(t * C, C)], sem_b))

    def fire(t):
        sem = sem_g0 if t % 2 == 0 else sem_g1
        return [pltpu.async_copy(uf_hbm.at[uidx[t]], ubuf[t % 2], sem),
                pltpu.async_copy(if_hbm.at[iidx[t]], ibuf[t % 2], sem)]

    def compute(t):
        ub_, ib_ = ubuf[t % 2], ibuf[t % 2]

        @plsc.parallel_loop(0, C, step=1, unroll=4)
        def pair_body(p):
            prods = [ub_[p, pl.ds(j * 16, 16)] * ib_[p, pl.ds(j * 16, 16)]
                     for j in range(8)]
            acc = ((prods[0] + prods[1]) + (prods[2] + prods[3])) \
                + ((prods[4] + prods[5]) + (prods[6] + prods[7]))
            pos = jnp.full((16,), t * C + p, jnp.int32)
            plsc.addupdate_scatter(outv, [pos], acc)

    inflight = fire(0)

    for cp in b_cp:
        cp.wait()

    def bias_group(g, carry):
        s = pl.ds(g * 16, 16)
        outv[s] = buv[s] + biv[s]
        return carry

    lax.fori_loop(0, PW // 16, bias_group, 0)

    for t in range(NCHUNK):
        nxt = fire(t + 1) if t + 1 < NCHUNK else []
        for cp in inflight:
            cp.wait()
        compute(t)
        inflight = nxt

    pltpu.sync_copy(outv, out_hbm.at[pl.ds(base, PW)])


@jax.jit
def kernel(u, i, user_f, item_f, user_b, item_b, global_b):
    u2 = u.astype(jnp.int32).reshape(NW, NCHUNK, C)
    i2 = i.astype(jnp.int32).reshape(NW, NCHUNK, C)
    ub = user_b.reshape(-1)
    ib = item_b.reshape(-1)
    mesh = plsc.VectorSubcoreMesh(core_axis_name="c", subcore_axis_name="s")
    fn = pl.kernel(
        _mf_body,
        out_type=jax.ShapeDtypeStruct((B,), jnp.float32),
        mesh=mesh,
        scratch_types=[pltpu.VMEM((C,), jnp.int32)] * 8
        + [pltpu.VMEM((C, K), jnp.float32)] * 4
        + [pltpu.VMEM((PW,), jnp.float32)] * 3
        + [pltpu.SemaphoreType.DMA] * 4,
        compiler_params=pltpu.CompilerParams(needs_layout_passes=False),
    )
    out = fn(u2, i2, user_f, item_f, ub, ib)
    return out + global_b
